# baseline (speedup 1.0000x reference)
import jax
import jax.numpy as jnp
from jax.experimental import pallas as pl
from jax.experimental.pallas import tpu as pltpu
from jax.experimental.pallas import tpu_sc as plsc

_SMOOTHING = 0.1
_N = 32000
_B = 2048
_CONF = 1.0 - _SMOOTHING
_SV = _SMOOTHING / (_N - 1)

_R = 64
_NI = _B // _R

_W = 512
_SCR = 8
_NLANE = 16

_RF = 256
_CF = 3200
_NIF = _B // _RF
_NJF = _N // _CF

_L = 128
_NK = _N // _L
_LOG2E = 1.4426950408889634


def _stream_body(x_ref, lse_ref, S_ref):
    for h in range(2):
        rows = pl.ds(h * (_R // 2), _R // 2)
        chunk0 = x_ref[rows, 0:_L]
        m = chunk0
        Ss = chunk0
        for k in range(1, _NK):
            xk = x_ref[rows, k * _L:(k + 1) * _L]
            m = jnp.maximum(m, xk)
            Ss = Ss + xk
        mm = m * _LOG2E
        s = jnp.exp2(chunk0 * _LOG2E - mm)
        for k in range(1, _NK):
            s = s + jnp.exp2(x_ref[rows, k * _L:(k + 1) * _L] * _LOG2E - mm)

        m_row = jnp.max(m, axis=1, keepdims=True)
        s_row = jnp.sum(s * jnp.exp(m - m_row), axis=1, keepdims=True)
        lse_ref[rows, :] = m_row + jnp.log(s_row)
        S_ref[rows, :] = jnp.sum(Ss, axis=1, keepdims=True)


def _sc_window(target, output, iota16):
    mesh = plsc.VectorSubcoreMesh(core_axis_name="c", subcore_axis_name="s")

    @pl.kernel(
        out_type=[
            jax.ShapeDtypeStruct((_B, _NLANE), jnp.int32),
            jax.ShapeDtypeStruct((_B, _NLANE), jnp.float32),
        ],
        mesh=mesh,
        scratch_types=[
            pltpu.VMEM((1, _NLANE), jnp.int32),
            pltpu.VMEM((1, _NLANE), jnp.float32),
        ],
    )
    def sc_kernel(t_hbm, x_hbm, io_hbm, ci_hbm, cv_hbm, cand_ref, cvl_ref):
        def body(t_vmem, x_vmem, io_vmem, ci_vmem, cv_vmem):
            @pl.loop(0, _SCR)
            def _row(r):
                cand_ref[...] = jnp.full((1, _NLANE), _N, jnp.int32)
                cvl_ref[...] = x_vmem[pl.ds(r, 1), 0:_NLANE]

                @pl.loop(0, _W // _NLANE)
                def _chunk(k):
                    sl = pl.ds(k * _NLANE, _NLANE)
                    tk = t_vmem[pl.ds(r, 1), sl]
                    xk = x_vmem[pl.ds(r, 1), sl]
                    idx = io_vmem[...] + k * _NLANE
                    cand = cand_ref[...]
                    take = jnp.logical_and(tk == 1, cand == _N)
                    cand_ref[...] = jnp.where(take, idx, cand)
                    cvl_ref[...] = jnp.where(take, xk, cvl_ref[...])

                ci_vmem[pl.ds(r, 1), :] = cand_ref[...]
                cv_vmem[pl.ds(r, 1), :] = cvl_ref[...]

        pltpu.emit_pipeline(
            body,
            grid=(_B // _SCR,),
            in_specs=[
                pl.BlockSpec((_SCR, _W), lambda i: (i, 0)),
                pl.BlockSpec((_SCR, _W), lambda i: (i, 0)),
                pl.BlockSpec((1, _NLANE), lambda i: (0, 0)),
            ],
            out_specs=[
                pl.BlockSpec((_SCR, _NLANE), lambda i: (i, 0)),
                pl.BlockSpec((_SCR, _NLANE), lambda i: (i, 0)),
            ],
            core_axis_name=("c", "s"),
            dimension_semantics=(pltpu.PARALLEL,),
        )(t_hbm, x_hbm, io_hbm, ci_hbm, cv_hbm)

    return sc_kernel(target, output, iota16)


def _lanered_body(ci_ref, cv_ref, cidx_ref, cval_ref):
    ci = ci_ref[...]
    cv = cv_ref[...]
    cidx = jnp.min(ci, axis=1, keepdims=True)
    cval = jnp.max(jnp.where(ci == cidx, cv, -jnp.inf), axis=1, keepdims=True)
    cidx_ref[...] = cidx
    cval_ref[...] = jnp.where(cidx == _N, cv[:, 0:1], cval)


def _fallback_body(x_ref, t_ref, cval_ref, ci_ref, cv_ref):
    j = pl.program_id(1)
    x = x_ref[...]
    t = t_ref[...]

    @pl.when(j == 0)
    def _init():
        ci_ref[...] = jnp.full((_RF, 1), _N, jnp.int32)
        cv_ref[...] = x[:, 0:1]

    iota = jax.lax.broadcasted_iota(jnp.int32, (_RF, _CF), 1) + j * _CF
    cand = jnp.where(t == 1, iota, _N)
    bidx = jnp.min(cand, axis=1, keepdims=True)
    bval = jnp.max(jnp.where(cand == bidx, x, -jnp.inf), axis=1, keepdims=True)
    take = bidx < ci_ref[...]
    ci_ref[...] = jnp.where(take, bidx, ci_ref[...])
    cv_ref[...] = jnp.where(take, bval, cv_ref[...])

    @pl.when(j == _NJF - 1)
    def _fin():
        cval_ref[...] = cv_ref[...]


def _combine_body(lse_ref, S_ref, cval_ref, out_ref):
    lse = lse_ref[...]
    S = S_ref[...]
    cval = cval_ref[...]
    rowloss = _SV * (_N * lse - S) + (_CONF - _SV) * (lse - cval)
    out_ref[...] = jnp.sum(rowloss, axis=(0, 1), keepdims=True) / _B


def _fallback_call(output, target):
    return pl.pallas_call(
        _fallback_body,
        grid=(_NIF, _NJF),
        in_specs=[
            pl.BlockSpec((_RF, _CF), lambda i, j: (i, j)),
            pl.BlockSpec((_RF, _CF), lambda i, j: (i, j)),
        ],
        out_specs=pl.BlockSpec((_RF, 1), lambda i, j: (i, 0)),
        out_shape=jax.ShapeDtypeStruct((_B, 1), jnp.float32),
        scratch_shapes=[
            pltpu.VMEM((_RF, 1), jnp.int32),
            pltpu.VMEM((_RF, 1), jnp.float32),
        ],
    )(output, target)


def kernel(output, target):
    iota16 = jax.lax.iota(jnp.int32, _NLANE).reshape(1, _NLANE)
    ci_lanes, cv_lanes = _sc_window(target, output, iota16)

    lse, S = pl.pallas_call(
        _stream_body,
        grid=(_NI,),
        in_specs=[pl.BlockSpec((_R, _N), lambda i: (i, 0))],
        out_specs=[
            pl.BlockSpec((_R, 1), lambda i: (i, 0)),
            pl.BlockSpec((_R, 1), lambda i: (i, 0)),
        ],
        out_shape=[
            jax.ShapeDtypeStruct((_B, 1), jnp.float32),
            jax.ShapeDtypeStruct((_B, 1), jnp.float32),
        ],
    )(output)

    cidx, cval = pl.pallas_call(
        _lanered_body,
        out_shape=[
            jax.ShapeDtypeStruct((_B, 1), jnp.int32),
            jax.ShapeDtypeStruct((_B, 1), jnp.float32),
        ],
    )(ci_lanes, cv_lanes)

    unresolved = jnp.any(cidx == _N)
    cval = jax.lax.cond(
        unresolved,
        lambda o, t, cv: _fallback_call(o, t),
        lambda o, t, cv: cv,
        output, target, cval,
    )

    loss = pl.pallas_call(
        _combine_body,
        out_shape=jax.ShapeDtypeStruct((1, 1), jnp.float32),
    )(lse, S, cval)
    return loss[0, 0]


# speedup vs baseline: 1.0070x; 1.0070x over previous
import jax
import jax.numpy as jnp
from jax.experimental import pallas as pl
from jax.experimental.pallas import tpu as pltpu
from jax.experimental.pallas import tpu_sc as plsc

_SMOOTHING = 0.1
_N = 32000
_B = 2048
_CONF = 1.0 - _SMOOTHING
_SV = _SMOOTHING / (_N - 1)

_R = 64
_NI = _B // _R

_W = 256
_SCR = 8
_NLANE = 16

_RF = 256
_CF = 3200
_NIF = _B // _RF
_NJF = _N // _CF

_L = 128
_NK = _N // _L
_LOG2E = 1.4426950408889634


def _stream_body(x_ref, lse_ref, S_ref):
    for h in range(2):
        rows = pl.ds(h * (_R // 2), _R // 2)
        chunk0 = x_ref[rows, 0:_L]
        m = chunk0
        Ss = chunk0
        for k in range(1, _NK):
            xk = x_ref[rows, k * _L:(k + 1) * _L]
            m = jnp.maximum(m, xk)
            Ss = Ss + xk
        mm = m * _LOG2E
        s = jnp.exp2(chunk0 * _LOG2E - mm)
        for k in range(1, _NK):
            s = s + jnp.exp2(x_ref[rows, k * _L:(k + 1) * _L] * _LOG2E - mm)

        m_row = jnp.max(m, axis=1, keepdims=True)
        s_row = jnp.sum(s * jnp.exp(m - m_row), axis=1, keepdims=True)
        lse_ref[rows, :] = m_row + jnp.log(s_row)
        S_ref[rows, :] = jnp.sum(Ss, axis=1, keepdims=True)


def _sc_window(target, output, iota16):
    mesh = plsc.VectorSubcoreMesh(core_axis_name="c", subcore_axis_name="s")

    @pl.kernel(
        out_type=[
            jax.ShapeDtypeStruct((_B, _NLANE), jnp.int32),
            jax.ShapeDtypeStruct((_B, _NLANE), jnp.float32),
        ],
        mesh=mesh,
        scratch_types=[
            pltpu.VMEM((1, _NLANE), jnp.int32),
            pltpu.VMEM((1, _NLANE), jnp.float32),
        ],
    )
    def sc_kernel(t_hbm, x_hbm, io_hbm, ci_hbm, cv_hbm, cand_ref, cvl_ref):
        def body(t_vmem, x_vmem, io_vmem, ci_vmem, cv_vmem):
            @pl.loop(0, _SCR)
            def _row(r):
                cand_ref[...] = jnp.full((1, _NLANE), _N, jnp.int32)
                cvl_ref[...] = x_vmem[pl.ds(r, 1), 0:_NLANE]

                @pl.loop(0, _W // _NLANE)
                def _chunk(k):
                    sl = pl.ds(k * _NLANE, _NLANE)
                    tk = t_vmem[pl.ds(r, 1), sl]
                    xk = x_vmem[pl.ds(r, 1), sl]
                    idx = io_vmem[...] + k * _NLANE
                    cand = cand_ref[...]
                    take = jnp.logical_and(tk == 1, cand == _N)
                    cand_ref[...] = jnp.where(take, idx, cand)
                    cvl_ref[...] = jnp.where(take, xk, cvl_ref[...])

                ci_vmem[pl.ds(r, 1), :] = cand_ref[...]
                cv_vmem[pl.ds(r, 1), :] = cvl_ref[...]

        pltpu.emit_pipeline(
            body,
            grid=(_B // _SCR,),
            in_specs=[
                pl.BlockSpec((_SCR, _W), lambda i: (i, 0)),
                pl.BlockSpec((_SCR, _W), lambda i: (i, 0)),
                pl.BlockSpec((1, _NLANE), lambda i: (0, 0)),
            ],
            out_specs=[
                pl.BlockSpec((_SCR, _NLANE), lambda i: (i, 0)),
                pl.BlockSpec((_SCR, _NLANE), lambda i: (i, 0)),
            ],
            core_axis_name=("c", "s"),
            dimension_semantics=(pltpu.PARALLEL,),
        )(t_hbm, x_hbm, io_hbm, ci_hbm, cv_hbm)

    return sc_kernel(target, output, iota16)


def _lanered_body(ci_ref, cv_ref, cidx_ref, cval_ref):
    ci = ci_ref[...]
    cv = cv_ref[...]
    cidx = jnp.min(ci, axis=1, keepdims=True)
    cval = jnp.max(jnp.where(ci == cidx, cv, -jnp.inf), axis=1, keepdims=True)
    cidx_ref[...] = cidx
    cval_ref[...] = jnp.where(cidx == _N, cv[:, 0:1], cval)


def _fallback_body(x_ref, t_ref, cval_ref, ci_ref, cv_ref):
    j = pl.program_id(1)
    x = x_ref[...]
    t = t_ref[...]

    @pl.when(j == 0)
    def _init():
        ci_ref[...] = jnp.full((_RF, 1), _N, jnp.int32)
        cv_ref[...] = x[:, 0:1]

    iota = jax.lax.broadcasted_iota(jnp.int32, (_RF, _CF), 1) + j * _CF
    cand = jnp.where(t == 1, iota, _N)
    bidx = jnp.min(cand, axis=1, keepdims=True)
    bval = jnp.max(jnp.where(cand == bidx, x, -jnp.inf), axis=1, keepdims=True)
    take = bidx < ci_ref[...]
    ci_ref[...] = jnp.where(take, bidx, ci_ref[...])
    cv_ref[...] = jnp.where(take, bval, cv_ref[...])

    @pl.when(j == _NJF - 1)
    def _fin():
        cval_ref[...] = cv_ref[...]


def _combine_body(lse_ref, S_ref, cval_ref, out_ref):
    lse = lse_ref[...]
    S = S_ref[...]
    cval = cval_ref[...]
    rowloss = _SV * (_N * lse - S) + (_CONF - _SV) * (lse - cval)
    out_ref[...] = jnp.sum(rowloss, axis=(0, 1), keepdims=True) / _B


def _fallback_call(output, target):
    return pl.pallas_call(
        _fallback_body,
        grid=(_NIF, _NJF),
        in_specs=[
            pl.BlockSpec((_RF, _CF), lambda i, j: (i, j)),
            pl.BlockSpec((_RF, _CF), lambda i, j: (i, j)),
        ],
        out_specs=pl.BlockSpec((_RF, 1), lambda i, j: (i, 0)),
        out_shape=jax.ShapeDtypeStruct((_B, 1), jnp.float32),
        scratch_shapes=[
            pltpu.VMEM((_RF, 1), jnp.int32),
            pltpu.VMEM((_RF, 1), jnp.float32),
        ],
    )(output, target)


def kernel(output, target):
    lse, S = pl.pallas_call(
        _stream_body,
        grid=(_NI,),
        in_specs=[pl.BlockSpec((_R, _N), lambda i: (i, 0))],
        out_specs=[
            pl.BlockSpec((_R, 1), lambda i: (i, 0)),
            pl.BlockSpec((_R, 1), lambda i: (i, 0)),
        ],
        out_shape=[
            jax.ShapeDtypeStruct((_B, 1), jnp.float32),
            jax.ShapeDtypeStruct((_B, 1), jnp.float32),
        ],
    )(output)

    iota16 = jax.lax.iota(jnp.int32, _NLANE).reshape(1, _NLANE)
    ci_lanes, cv_lanes = _sc_window(target, output, iota16)

    cidx, cval = pl.pallas_call(
        _lanered_body,
        out_shape=[
            jax.ShapeDtypeStruct((_B, 1), jnp.int32),
            jax.ShapeDtypeStruct((_B, 1), jnp.float32),
        ],
    )(ci_lanes, cv_lanes)

    unresolved = jnp.any(cidx == _N)
    cval = jax.lax.cond(
        unresolved,
        lambda o, t, cv: _fallback_call(o, t),
        lambda o, t, cv: cv,
        output, target, cval,
    )

    loss = pl.pallas_call(
        _combine_body,
        out_shape=jax.ShapeDtypeStruct((1, 1), jnp.float32),
    )(lse, S, cval)
    return loss[0, 0]


# speedup vs baseline: 1.0094x; 1.0025x over previous
import jax
import jax.numpy as jnp
from jax.experimental import pallas as pl
from jax.experimental.pallas import tpu as pltpu
from jax.experimental.pallas import tpu_sc as plsc

_SMOOTHING = 0.1
_N = 32000
_B = 2048
_CONF = 1.0 - _SMOOTHING
_SV = _SMOOTHING / (_N - 1)

_R = 64
_NI = _B // _R

_W = 256
_SCR = 8
_NLANE = 16

_RF = 256
_CF = 3200
_NIF = _B // _RF
_NJF = _N // _CF

_L = 128
_NK = _N // _L
_LOG2E = 1.4426950408889634


def _stream_body(x_ref, lse_ref, S_ref):
    for h in range(2):
        rows = pl.ds(h * (_R // 2), _R // 2)
        chunk0 = x_ref[rows, 0:_L]
        m = chunk0
        Ss = chunk0
        for k in range(1, _NK):
            xk = x_ref[rows, k * _L:(k + 1) * _L]
            m = jnp.maximum(m, xk)
            Ss = Ss + xk
        mm = m * _LOG2E
        s = jnp.exp2(chunk0 * _LOG2E - mm)
        for k in range(1, _NK):
            s = s + jnp.exp2(x_ref[rows, k * _L:(k + 1) * _L] * _LOG2E - mm)

        m_row = jnp.max(m, axis=1, keepdims=True)
        s_row = jnp.sum(s * jnp.exp(m - m_row), axis=1, keepdims=True)
        lse_ref[rows, :] = m_row + jnp.log(s_row)
        S_ref[rows, :] = jnp.sum(Ss, axis=1, keepdims=True)


def _sc_window(target, iota16):
    mesh = plsc.VectorSubcoreMesh(core_axis_name="c", subcore_axis_name="s")

    @pl.kernel(
        out_type=jax.ShapeDtypeStruct((_B, _NLANE), jnp.int32),
        mesh=mesh,
        scratch_types=[
            pltpu.VMEM((1, _NLANE), jnp.int32),
        ],
    )
    def sc_kernel(t_hbm, io_hbm, ci_hbm, cand_ref):
        def body(t_vmem, io_vmem, ci_vmem):
            @pl.loop(0, _SCR)
            def _row(r):
                cand_ref[...] = jnp.full((1, _NLANE), _N, jnp.int32)

                @pl.loop(0, _W // _NLANE)
                def _chunk(k):
                    sl = pl.ds(k * _NLANE, _NLANE)
                    tk = t_vmem[pl.ds(r, 1), sl]
                    idx = io_vmem[...] + k * _NLANE
                    cand = cand_ref[...]
                    take = jnp.logical_and(tk == 1, cand == _N)
                    cand_ref[...] = jnp.where(take, idx, cand)

                ci_vmem[pl.ds(r, 1), :] = cand_ref[...]

        pltpu.emit_pipeline(
            body,
            grid=(_B // _SCR,),
            in_specs=[
                pl.BlockSpec((_SCR, _W), lambda i: (i, 0)),
                pl.BlockSpec((1, _NLANE), lambda i: (0, 0)),
            ],
            out_specs=[
                pl.BlockSpec((_SCR, _NLANE), lambda i: (i, 0)),
            ],
            core_axis_name=("c", "s"),
            dimension_semantics=(pltpu.PARALLEL,),
        )(t_hbm, io_hbm, ci_hbm)

    return sc_kernel(target, iota16)


def _lanered_body(ci_ref, xw_ref, cidx_ref, cval_ref):
    ci = ci_ref[...]
    xw = xw_ref[...]
    cidx = jnp.min(ci, axis=1, keepdims=True)
    iota = jax.lax.broadcasted_iota(jnp.int32, (_B, _W), 1)
    cval = jnp.max(jnp.where(iota == cidx, xw, -jnp.inf), axis=1,
                   keepdims=True)
    cidx_ref[...] = cidx
    cval_ref[...] = jnp.where(cidx == _N, xw[:, 0:1], cval)


def _fallback_body(x_ref, t_ref, cval_ref, ci_ref, cv_ref):
    j = pl.program_id(1)
    x = x_ref[...]
    t = t_ref[...]

    @pl.when(j == 0)
    def _init():
        ci_ref[...] = jnp.full((_RF, 1), _N, jnp.int32)
        cv_ref[...] = x[:, 0:1]

    iota = jax.lax.broadcasted_iota(jnp.int32, (_RF, _CF), 1) + j * _CF
    cand = jnp.where(t == 1, iota, _N)
    bidx = jnp.min(cand, axis=1, keepdims=True)
    bval = jnp.max(jnp.where(cand == bidx, x, -jnp.inf), axis=1, keepdims=True)
    take = bidx < ci_ref[...]
    ci_ref[...] = jnp.where(take, bidx, ci_ref[...])
    cv_ref[...] = jnp.where(take, bval, cv_ref[...])

    @pl.when(j == _NJF - 1)
    def _fin():
        cval_ref[...] = cv_ref[...]


def _combine_body(lse_ref, S_ref, cval_ref, out_ref):
    lse = lse_ref[...]
    S = S_ref[...]
    cval = cval_ref[...]
    rowloss = _SV * (_N * lse - S) + (_CONF - _SV) * (lse - cval)
    out_ref[...] = jnp.sum(rowloss, axis=(0, 1), keepdims=True) / _B


def _fallback_call(output, target):
    return pl.pallas_call(
        _fallback_body,
        grid=(_NIF, _NJF),
        in_specs=[
            pl.BlockSpec((_RF, _CF), lambda i, j: (i, j)),
            pl.BlockSpec((_RF, _CF), lambda i, j: (i, j)),
        ],
        out_specs=pl.BlockSpec((_RF, 1), lambda i, j: (i, 0)),
        out_shape=jax.ShapeDtypeStruct((_B, 1), jnp.float32),
        scratch_shapes=[
            pltpu.VMEM((_RF, 1), jnp.int32),
            pltpu.VMEM((_RF, 1), jnp.float32),
        ],
    )(output, target)


def kernel(output, target):
    lse, S = pl.pallas_call(
        _stream_body,
        grid=(_NI,),
        in_specs=[pl.BlockSpec((_R, _N), lambda i: (i, 0))],
        out_specs=[
            pl.BlockSpec((_R, 1), lambda i: (i, 0)),
            pl.BlockSpec((_R, 1), lambda i: (i, 0)),
        ],
        out_shape=[
            jax.ShapeDtypeStruct((_B, 1), jnp.float32),
            jax.ShapeDtypeStruct((_B, 1), jnp.float32),
        ],
    )(output)

    iota16 = jax.lax.iota(jnp.int32, _NLANE).reshape(1, _NLANE)
    ci_lanes = _sc_window(target, iota16)

    cidx, cval = pl.pallas_call(
        _lanered_body,
        grid=(1,),
        in_specs=[
            pl.BlockSpec((_B, _NLANE), lambda i: (0, 0)),
            pl.BlockSpec((_B, _W), lambda i: (0, 0)),
        ],
        out_specs=[
            pl.BlockSpec((_B, 1), lambda i: (0, 0)),
            pl.BlockSpec((_B, 1), lambda i: (0, 0)),
        ],
        out_shape=[
            jax.ShapeDtypeStruct((_B, 1), jnp.int32),
            jax.ShapeDtypeStruct((_B, 1), jnp.float32),
        ],
    )(ci_lanes, output)

    unresolved = jnp.any(cidx == _N)
    cval = jax.lax.cond(
        unresolved,
        lambda o, t, cv: _fallback_call(o, t),
        lambda o, t, cv: cv,
        output, target, cval,
    )

    loss = pl.pallas_call(
        _combine_body,
        out_shape=jax.ShapeDtypeStruct((1, 1), jnp.float32),
    )(lse, S, cval)
    return loss[0, 0]


# speedup vs baseline: 1.1932x; 1.1821x over previous
import jax
import jax.numpy as jnp
from jax.experimental import pallas as pl
from jax.experimental.pallas import tpu as pltpu

_SMOOTHING = 0.1
_N = 32000
_B = 2048
_CONF = 1.0 - _SMOOTHING
_SV = _SMOOTHING / (_N - 1)

_R = 64
_W = 640
_NI = _B // _R

_RF = 256
_CF = 3200
_NIF = _B // _RF
_NJF = _N // _CF

_L = 128
_NK = _N // _L
_LOG2E = 1.4426950408889634


def _stream_body(x_ref, t_ref, lse_ref, S_ref, cval_ref, cidx_ref):
    t = t_ref[...]
    xw = x_ref[:, :_W]
    iota = jax.lax.broadcasted_iota(jnp.int32, (_R, _W), 1)
    cand = jnp.where(t == 1, iota, _N)
    cidx = jnp.min(cand, axis=1, keepdims=True)
    cval = jnp.max(jnp.where(cand == cidx, xw, -jnp.inf), axis=1,
                   keepdims=True)
    cidx_ref[...] = cidx
    cval_ref[...] = jnp.where(cidx == _N, x_ref[:, 0:1], cval)

    for h in range(2):
        rows = pl.ds(h * (_R // 2), _R // 2)
        chunk0 = x_ref[rows, 0:_L]
        m = chunk0
        Ss = chunk0
        for k in range(1, _NK):
            xk = x_ref[rows, k * _L:(k + 1) * _L]
            m = jnp.maximum(m, xk)
            Ss = Ss + xk
        mm = m * _LOG2E
        s = jnp.exp2(chunk0 * _LOG2E - mm)
        for k in range(1, _NK):
            s = s + jnp.exp2(x_ref[rows, k * _L:(k + 1) * _L] * _LOG2E - mm)

        m_row = jnp.max(m, axis=1, keepdims=True)
        s_row = jnp.sum(s * jnp.exp(m - m_row), axis=1, keepdims=True)
        lse_ref[rows, :] = m_row + jnp.log(s_row)
        S_ref[rows, :] = jnp.sum(Ss, axis=1, keepdims=True)


def _fallback_body(x_ref, t_ref, cval_ref, ci_ref, cv_ref):
    j = pl.program_id(1)
    x = x_ref[...]
    t = t_ref[...]

    @pl.when(j == 0)
    def _init():
        ci_ref[...] = jnp.full((_RF, 1), _N, jnp.int32)
        cv_ref[...] = x[:, 0:1]

    iota = jax.lax.broadcasted_iota(jnp.int32, (_RF, _CF), 1) + j * _CF
    cand = jnp.where(t == 1, iota, _N)
    bidx = jnp.min(cand, axis=1, keepdims=True)
    bval = jnp.max(jnp.where(cand == bidx, x, -jnp.inf), axis=1, keepdims=True)
    take = bidx < ci_ref[...]
    ci_ref[...] = jnp.where(take, bidx, ci_ref[...])
    cv_ref[...] = jnp.where(take, bval, cv_ref[...])

    @pl.when(j == _NJF - 1)
    def _fin():
        cval_ref[...] = cv_ref[...]


def _combine_body(lse_ref, S_ref, cval_ref, out_ref):
    lse = lse_ref[...]
    S = S_ref[...]
    cval = cval_ref[...]
    rowloss = _SV * (_N * lse - S) + (_CONF - _SV) * (lse - cval)
    out_ref[...] = jnp.sum(rowloss, axis=(0, 1), keepdims=True) / _B


def _fallback_call(output, target):
    return pl.pallas_call(
        _fallback_body,
        grid=(_NIF, _NJF),
        in_specs=[
            pl.BlockSpec((_RF, _CF), lambda i, j: (i, j)),
            pl.BlockSpec((_RF, _CF), lambda i, j: (i, j)),
        ],
        out_specs=pl.BlockSpec((_RF, 1), lambda i, j: (i, 0)),
        out_shape=jax.ShapeDtypeStruct((_B, 1), jnp.float32),
        scratch_shapes=[
            pltpu.VMEM((_RF, 1), jnp.int32),
            pltpu.VMEM((_RF, 1), jnp.float32),
        ],
    )(output, target)


def kernel(output, target):
    lse, S, cval, cidx = pl.pallas_call(
        _stream_body,
        grid=(_NI,),
        in_specs=[
            pl.BlockSpec((_R, _N), lambda i: (i, 0)),
            pl.BlockSpec((_R, _W), lambda i: (i, 0)),
        ],
        out_specs=[
            pl.BlockSpec((_R, 1), lambda i: (i, 0)),
            pl.BlockSpec((_R, 1), lambda i: (i, 0)),
            pl.BlockSpec((_R, 1), lambda i: (i, 0)),
            pl.BlockSpec((_R, 1), lambda i: (i, 0)),
        ],
        out_shape=[
            jax.ShapeDtypeStruct((_B, 1), jnp.float32),
            jax.ShapeDtypeStruct((_B, 1), jnp.float32),
            jax.ShapeDtypeStruct((_B, 1), jnp.float32),
            jax.ShapeDtypeStruct((_B, 1), jnp.int32),
        ],
    )(output, target)

    unresolved = jnp.any(cidx == _N)
    cval = jax.lax.cond(
        unresolved,
        lambda o, t, cv: _fallback_call(o, t),
        lambda o, t, cv: cv,
        output, target, cval,
    )

    loss = pl.pallas_call(
        _combine_body,
        out_shape=jax.ShapeDtypeStruct((1, 1), jnp.float32),
    )(lse, S, cval)
    return loss[0, 0]
